# baseline (speedup 1.0000x reference)
import jax
import jax.numpy as jnp
from jax import lax
from jax.experimental import pallas as pl
from jax.experimental.pallas import tpu as pltpu
from jax.experimental.pallas import tpu_sc as plsc

NC = 2
NS = 16
NW = NC * NS

B = 16384
EMB = 32
OUT_D = 4 * EMB
SAMP = B // NW

GP = 80
GC = 64

SUBJ_T, SUBJ_SPC = 20, 32
DESC_T, DESC_SPC = 200, 4
TYPE_T, TYPE_SPC = 20, 32

MAX_ROWS = DESC_SPC * DESC_T
MAX_IDX = MAX_ROWS // GP
NACC = 4
UNROLL = 40
GRP = 8


def _accum_chunk(rows, outbuf, chunk, col, T, spc):
    inv = jnp.float32(1.0 / T)
    zero = jnp.zeros((16,), jnp.float32)
    grouped = T % UNROLL == 0
    u = UNROLL if grouped else T

    @pl.loop(0, spc)
    def samp_loop(s):
        rb = s * T

        def tok(jo, carry):
            lo = list(carry[:NACC])
            hi = list(carry[NACC:])
            base = rb + jo * u
            if grouped:
                for grp in range(u // GRP):
                    g0 = base + grp * GRP
                    gs = rows[g0]
                    for j in range(1, GRP):
                        gs = gs + rows[g0 + j]
                    a, b = plsc.unpack(gs,
                                       format=plsc.PackFormat.INTERLEAVED)
                    k = grp % NACC
                    lo[k] = lo[k] + a
                    hi[k] = hi[k] + b
            else:
                for j in range(u):
                    a, b = plsc.unpack(rows[base + j],
                                       format=plsc.PackFormat.INTERLEAVED)
                    k = j % NACC
                    lo[k] = lo[k] + a
                    hi[k] = hi[k] + b
            return tuple(lo) + tuple(hi)

        acc = lax.fori_loop(0, T // u, tok, (zero,) * (2 * NACC))
        a0 = (acc[0] + acc[1]) + (acc[2] + acc[3])
        a1 = (acc[4] + acc[5]) + (acc[6] + acc[7])
        orow = chunk * spc + s
        outbuf[orow, pl.ds(col, 16)] = a0 * inv
        outbuf[orow, pl.ds(col + 16, 16)] = a1 * inv


def _pooled_field(wid, idx_hbm, tab_hbm, col,
                  idx_bufs, rows_bufs, outbuf, sem_rows, sem_idx, T, spc):
    nrc = spc * T
    nchunks = SAMP // spc
    tbase = wid * SAMP * T

    def fetch_idx(chunk, b):
        pltpu.async_copy(idx_hbm.at[pl.ds(tbase + chunk * nrc, nrc)],
                         idx_bufs[b].at[pl.ds(0, nrc)], sem_idx[b])

    def wait_idx(b):
        pltpu.make_async_copy(idx_hbm.at[pl.ds(0, nrc)],
                              idx_bufs[b].at[pl.ds(0, nrc)], sem_idx[b]).wait()

    def fire_gathers(b):
        pltpu.async_copy(tab_hbm.at[idx_bufs[b].at[pl.ds(0, nrc)]],
                         rows_bufs[b].at[pl.ds(0, nrc)], sem_rows[b])

    def wait_gathers(b):
        pltpu.make_async_copy(tab_hbm.at[pl.ds(0, nrc)],
                              rows_bufs[b].at[pl.ds(0, nrc)], sem_rows[b]).wait()

    pltpu.sync_copy(idx_hbm.at[pl.ds(tbase, nrc)],
                    idx_bufs[0].at[pl.ds(0, nrc)])
    fire_gathers(0)
    fetch_idx(1, 1)

    @pl.loop(0, nchunks - 2, step=2)
    def chunk_loop(i):
        for db in (0, 1):
            b = db
            ii = i + db
            wait_gathers(b)
            fetch_idx(ii + 2, b)
            wait_idx(1 - b)
            fire_gathers(1 - b)
            _accum_chunk(rows_bufs[b], outbuf, ii, col, T, spc)

    wait_gathers(0)
    wait_idx(1)
    fire_gathers(1)
    _accum_chunk(rows_bufs[0], outbuf, nchunks - 2, col, T, spc)
    wait_gathers(1)
    _accum_chunk(rows_bufs[1], outbuf, nchunks - 1, col, T, spc)


def _body(cust_i, subj_i, desc_i, type_i, ctab, stab, dtab, ttab, out,
          idx_c, idx_b0, idx_b1, rows_b0, rows_b1, outbuf,
          sem_r0, sem_r1, sem_i0, sem_i1):
    wid = lax.axis_index("s") * NC + lax.axis_index("c")
    idx_bufs = (idx_b0, idx_b1)
    rows_bufs = (rows_b0, rows_b1)
    sem_rows = (sem_r0, sem_r1)
    sem_idx = (sem_i0, sem_i1)

    pltpu.sync_copy(cust_i.at[pl.ds(wid * SAMP, SAMP)], idx_c)
    pltpu.async_copy(ctab.at[idx_c],
                     rows_b0.at[pl.ds(0, SAMP)], sem_r0).wait()

    @pl.loop(0, SAMP)
    def cust_loop(s):
        a, b = plsc.unpack(rows_b0[s], format=plsc.PackFormat.INTERLEAVED)
        outbuf[s, pl.ds(0, 16)] = a
        outbuf[s, pl.ds(16, 16)] = b

    _pooled_field(wid, subj_i, stab, EMB, idx_bufs, rows_bufs, outbuf,
                  sem_rows, sem_idx, SUBJ_T, SUBJ_SPC)
    _pooled_field(wid, desc_i, dtab, 2 * EMB, idx_bufs, rows_bufs, outbuf,
                  sem_rows, sem_idx, DESC_T, DESC_SPC)
    _pooled_field(wid, type_i, ttab, 3 * EMB, idx_bufs, rows_bufs, outbuf,
                  sem_rows, sem_idx, TYPE_T, TYPE_SPC)

    pltpu.sync_copy(outbuf, out.at[pl.ds(wid * SAMP, SAMP)])


@jax.jit
def _run(cust_i, subj_i, desc_i, type_i, ctab, stab, dtab, ttab):
    mesh = plsc.VectorSubcoreMesh(core_axis_name="c", subcore_axis_name="s",
                                  num_cores=NC, num_subcores=NS)
    fn = pl.kernel(
        _body,
        out_type=jax.ShapeDtypeStruct((B, OUT_D), jnp.float32),
        mesh=mesh,
        compiler_params=pltpu.CompilerParams(use_tc_tiling_on_sc=False,
                                             needs_layout_passes=False),
        scratch_types=[
            pltpu.VMEM((SAMP,), jnp.int32),
            pltpu.VMEM((MAX_ROWS,), jnp.int32),
            pltpu.VMEM((MAX_ROWS,), jnp.int32),
            pltpu.VMEM((MAX_ROWS, EMB), jnp.bfloat16),
            pltpu.VMEM((MAX_ROWS, EMB), jnp.bfloat16),
            pltpu.VMEM((SAMP, OUT_D), jnp.float32),
            pltpu.SemaphoreType.DMA,
            pltpu.SemaphoreType.DMA,
            pltpu.SemaphoreType.DMA,
            pltpu.SemaphoreType.DMA,
        ],
    )
    return fn(cust_i, subj_i, desc_i, type_i, ctab, stab, dtab, ttab)


def _prep_table(tab):
    v = tab.shape[0]
    half = EMB // 2
    inter = jnp.stack([tab[:, :half], tab[:, half:]], axis=-1)
    return inter.reshape(v, EMB).astype(jnp.bfloat16)


def kernel(customer_name, ticket_subject, ticket_description, ticket_type,
           customer_table, subject_table, description_table, type_table):
    cust_i = customer_name
    subj_i = ticket_subject.reshape(-1)
    desc_i = ticket_description.reshape(-1)
    type_i = ticket_type.reshape(-1)
    return _run(cust_i, subj_i, desc_i, type_i,
                _prep_table(customer_table), _prep_table(subject_table),
                _prep_table(description_table), _prep_table(type_table))


# speedup vs baseline: 1.4588x; 1.4588x over previous
import jax
import jax.numpy as jnp
from jax import lax
from jax.experimental import pallas as pl
from jax.experimental.pallas import tpu as pltpu
from jax.experimental.pallas import tpu_sc as plsc

NC = 2
NS = 16
NW = NC * NS

B = 16384
EMB = 32
OUT_D = 4 * EMB
VOCAB = 10000
SAMP = B // NW

GP = 80
GC = 64

SUBJ_T, SUBJ_SPC = 20, 32
DESC_T, DESC_SPC = 200, 4
TYPE_T, TYPE_SPC = 20, 32

MAX_ROWS = DESC_SPC * DESC_T
MAX_IDX = MAX_ROWS // GP
NACC = 4
UNROLL = 40
GRP = 8


def _accum_chunk(rows, outbuf, chunk, col, T, spc):
    inv = jnp.float32(1.0 / T)
    zero = jnp.zeros((16,), jnp.float32)
    grouped = T % UNROLL == 0
    u = UNROLL if grouped else T

    @pl.loop(0, spc)
    def samp_loop(s):
        rb = s * T

        def tok(jo, carry):
            lo = list(carry[:NACC])
            hi = list(carry[NACC:])
            base = rb + jo * u
            if grouped:
                for grp in range(u // GRP):
                    g0 = base + grp * GRP
                    gs = rows[g0]
                    for j in range(1, GRP):
                        gs = gs + rows[g0 + j]
                    a, b = plsc.unpack(gs,
                                       format=plsc.PackFormat.INTERLEAVED)
                    k = grp % NACC
                    lo[k] = lo[k] + a
                    hi[k] = hi[k] + b
            else:
                for j in range(u):
                    a, b = plsc.unpack(rows[base + j],
                                       format=plsc.PackFormat.INTERLEAVED)
                    k = j % NACC
                    lo[k] = lo[k] + a
                    hi[k] = hi[k] + b
            return tuple(lo) + tuple(hi)

        acc = lax.fori_loop(0, T // u, tok, (zero,) * (2 * NACC))
        a0 = (acc[0] + acc[1]) + (acc[2] + acc[3])
        a1 = (acc[4] + acc[5]) + (acc[6] + acc[7])
        orow = chunk * spc + s
        outbuf[orow, pl.ds(col, 16)] = a0 * inv
        outbuf[orow, pl.ds(col + 16, 16)] = a1 * inv


def _pooled_field(wid, idx_hbm, tab_hbm, col,
                  idx_bufs, rows_bufs, outbuf, sem_rows, sem_idx, T, spc):
    nrc = spc * T
    nchunks = SAMP // spc
    tbase = wid * SAMP * T

    def fetch_idx(chunk, b):
        pltpu.async_copy(idx_hbm.at[pl.ds(tbase + chunk * nrc, nrc)],
                         idx_bufs[b].at[pl.ds(0, nrc)], sem_idx[b])

    def wait_idx(b):
        pltpu.make_async_copy(idx_hbm.at[pl.ds(0, nrc)],
                              idx_bufs[b].at[pl.ds(0, nrc)], sem_idx[b]).wait()

    def fire_gathers(b):
        pltpu.async_copy(tab_hbm.at[idx_bufs[b].at[pl.ds(0, nrc)]],
                         rows_bufs[b].at[pl.ds(0, nrc)], sem_rows[b])

    def wait_gathers(b):
        pltpu.make_async_copy(tab_hbm.at[pl.ds(0, nrc)],
                              rows_bufs[b].at[pl.ds(0, nrc)], sem_rows[b]).wait()

    pltpu.sync_copy(idx_hbm.at[pl.ds(tbase, nrc)],
                    idx_bufs[0].at[pl.ds(0, nrc)])
    fire_gathers(0)
    fetch_idx(1, 1)

    @pl.loop(0, nchunks - 2, step=2)
    def chunk_loop(i):
        for db in (0, 1):
            b = db
            ii = i + db
            wait_gathers(b)
            fetch_idx(ii + 2, b)
            wait_idx(1 - b)
            fire_gathers(1 - b)
            _accum_chunk(rows_bufs[b], outbuf, ii, col, T, spc)

    wait_gathers(0)
    wait_idx(1)
    fire_gathers(1)
    _accum_chunk(rows_bufs[0], outbuf, nchunks - 2, col, T, spc)
    wait_gathers(1)
    _accum_chunk(rows_bufs[1], outbuf, nchunks - 1, col, T, spc)


def _body(cust_i, subj_i, desc_i, type_i, ctab, stab, dtab, ttab, out,
          idx_c, idx_b0, idx_b1, rows_b0, rows_b1, outbuf,
          sp_s, sp_d, sp_t,
          sem_r0, sem_r1, sem_i0, sem_i1):
    wid = lax.axis_index("s") * NC + lax.axis_index("c")
    idx_bufs = (idx_b0, idx_b1)
    rows_bufs = (rows_b0, rows_b1)
    sem_rows = (sem_r0, sem_r1)
    sem_idx = (sem_i0, sem_i1)

    sid = lax.axis_index("s")
    vrows = VOCAB // NS
    for tab_hbm, sp in ((stab, sp_s), (dtab, sp_d), (ttab, sp_t)):
        pltpu.sync_copy(tab_hbm.at[pl.ds(sid * vrows, vrows)],
                        sp.at[pl.ds(sid * vrows, vrows)])
    plsc.subcore_barrier()

    pltpu.sync_copy(cust_i.at[pl.ds(wid * SAMP, SAMP)], idx_c)
    pltpu.async_copy(ctab.at[idx_c],
                     rows_b0.at[pl.ds(0, SAMP)], sem_r0).wait()

    @pl.loop(0, SAMP)
    def cust_loop(s):
        a, b = plsc.unpack(rows_b0[s], format=plsc.PackFormat.INTERLEAVED)
        outbuf[s, pl.ds(0, 16)] = a
        outbuf[s, pl.ds(16, 16)] = b

    _pooled_field(wid, subj_i, sp_s, EMB, idx_bufs, rows_bufs, outbuf,
                  sem_rows, sem_idx, SUBJ_T, SUBJ_SPC)
    _pooled_field(wid, desc_i, sp_d, 2 * EMB, idx_bufs, rows_bufs, outbuf,
                  sem_rows, sem_idx, DESC_T, DESC_SPC)
    _pooled_field(wid, type_i, sp_t, 3 * EMB, idx_bufs, rows_bufs, outbuf,
                  sem_rows, sem_idx, TYPE_T, TYPE_SPC)

    pltpu.sync_copy(outbuf, out.at[pl.ds(wid * SAMP, SAMP)])


@jax.jit
def _run(cust_i, subj_i, desc_i, type_i, ctab, stab, dtab, ttab):
    mesh = plsc.VectorSubcoreMesh(core_axis_name="c", subcore_axis_name="s",
                                  num_cores=NC, num_subcores=NS)
    fn = pl.kernel(
        _body,
        out_type=jax.ShapeDtypeStruct((B, OUT_D), jnp.float32),
        mesh=mesh,
        compiler_params=pltpu.CompilerParams(use_tc_tiling_on_sc=False,
                                             needs_layout_passes=False),
        scratch_types=[
            pltpu.VMEM((SAMP,), jnp.int32),
            pltpu.VMEM((MAX_ROWS,), jnp.int32),
            pltpu.VMEM((MAX_ROWS,), jnp.int32),
            pltpu.VMEM((MAX_ROWS, EMB), jnp.bfloat16),
            pltpu.VMEM((MAX_ROWS, EMB), jnp.bfloat16),
            pltpu.VMEM((SAMP, OUT_D), jnp.float32),
            pltpu.VMEM_SHARED((VOCAB, EMB), jnp.bfloat16),
            pltpu.VMEM_SHARED((VOCAB, EMB), jnp.bfloat16),
            pltpu.VMEM_SHARED((VOCAB, EMB), jnp.bfloat16),
            pltpu.SemaphoreType.DMA,
            pltpu.SemaphoreType.DMA,
            pltpu.SemaphoreType.DMA,
            pltpu.SemaphoreType.DMA,
        ],
    )
    return fn(cust_i, subj_i, desc_i, type_i, ctab, stab, dtab, ttab)


def _prep_table(tab):
    v = tab.shape[0]
    half = EMB // 2
    inter = jnp.stack([tab[:, :half], tab[:, half:]], axis=-1)
    return inter.reshape(v, EMB).astype(jnp.bfloat16)


def kernel(customer_name, ticket_subject, ticket_description, ticket_type,
           customer_table, subject_table, description_table, type_table):
    cust_i = customer_name
    subj_i = ticket_subject.reshape(-1)
    desc_i = ticket_description.reshape(-1)
    type_i = ticket_type.reshape(-1)
    return _run(cust_i, subj_i, desc_i, type_i,
                _prep_table(customer_table), _prep_table(subject_table),
                _prep_table(description_table), _prep_table(type_table))


# speedup vs baseline: 1.7011x; 1.1661x over previous
import jax
import jax.numpy as jnp
from jax import lax
from jax.experimental import pallas as pl
from jax.experimental.pallas import tpu as pltpu
from jax.experimental.pallas import tpu_sc as plsc

NC = 2
NS = 16
NW = NC * NS

B = 16384
EMB = 32
OUT_D = 4 * EMB
VOCAB = 10000
SAMP = B // NW

GP = 80
GC = 64

SUBJ_T, SUBJ_SPC = 20, 32
DESC_T, DESC_SPC = 200, 4
TYPE_T, TYPE_SPC = 20, 32

MAX_ROWS = DESC_SPC * DESC_T
MAX_IDX = MAX_ROWS // GP
NACC = 4
UNROLL = 40
GRP = 8


def _accum_chunk(rows, outbuf, chunk, col, T, spc):
    inv = jnp.float32(1.0 / T)
    zero = jnp.zeros((16,), jnp.float32)
    grouped = T % UNROLL == 0
    u = UNROLL if grouped else T

    @pl.loop(0, spc)
    def samp_loop(s):
        rb = s * T

        def tok(jo, carry):
            lo = list(carry[:NACC])
            hi = list(carry[NACC:])
            base = rb + jo * u
            if grouped:
                for grp in range(u // GRP):
                    g0 = base + grp * GRP
                    gs = rows[g0]
                    for j in range(1, GRP):
                        gs = gs + rows[g0 + j]
                    a, b = plsc.unpack(gs,
                                       format=plsc.PackFormat.INTERLEAVED)
                    k = grp % NACC
                    lo[k] = lo[k] + a
                    hi[k] = hi[k] + b
            else:
                for j in range(u):
                    a, b = plsc.unpack(rows[base + j],
                                       format=plsc.PackFormat.INTERLEAVED)
                    k = j % NACC
                    lo[k] = lo[k] + a
                    hi[k] = hi[k] + b
            return tuple(lo) + tuple(hi)

        acc = lax.fori_loop(0, T // u, tok, (zero,) * (2 * NACC))
        a0 = (acc[0] + acc[1]) + (acc[2] + acc[3])
        a1 = (acc[4] + acc[5]) + (acc[6] + acc[7])
        orow = chunk * spc + s
        outbuf[orow, pl.ds(col, 16)] = a0 * inv
        outbuf[orow, pl.ds(col + 16, 16)] = a1 * inv


def _pooled_field(wid, idx_hbm, tab_hbm, col,
                  idx_bufs, rows_bufs, outbuf, sem_rows, sem_idx, T, spc):
    nrc = spc * T
    nchunks = SAMP // spc
    tbase = wid * SAMP * T

    def fetch_idx(chunk, b):
        pltpu.async_copy(idx_hbm.at[pl.ds(tbase + chunk * nrc, nrc)],
                         idx_bufs[b].at[pl.ds(0, nrc)], sem_idx[b])

    def wait_idx(b):
        pltpu.make_async_copy(idx_hbm.at[pl.ds(0, nrc)],
                              idx_bufs[b].at[pl.ds(0, nrc)], sem_idx[b]).wait()

    def fire_gathers(b):
        pltpu.async_copy(tab_hbm.at[idx_bufs[b].at[pl.ds(0, nrc)]],
                         rows_bufs[b].at[pl.ds(0, nrc)], sem_rows[b])

    def wait_gathers(b):
        pltpu.make_async_copy(tab_hbm.at[pl.ds(0, nrc)],
                              rows_bufs[b].at[pl.ds(0, nrc)], sem_rows[b]).wait()

    pltpu.sync_copy(idx_hbm.at[pl.ds(tbase, nrc)],
                    idx_bufs[0].at[pl.ds(0, nrc)])
    fire_gathers(0)
    fetch_idx(1, 1)

    @pl.loop(0, nchunks - 2, step=2)
    def chunk_loop(i):
        for db in (0, 1):
            b = db
            ii = i + db
            wait_gathers(b)
            fetch_idx(ii + 2, b)
            wait_idx(1 - b)
            fire_gathers(1 - b)
            _accum_chunk(rows_bufs[b], outbuf, ii, col, T, spc)

    wait_gathers(0)
    wait_idx(1)
    fire_gathers(1)
    _accum_chunk(rows_bufs[0], outbuf, nchunks - 2, col, T, spc)
    wait_gathers(1)
    _accum_chunk(rows_bufs[1], outbuf, nchunks - 1, col, T, spc)


def _body(cust_i, subj_i, desc_i, type_i, ctab, stab, dtab, ttab, out,
          idx_c, idx_b0, idx_b1, rows_b0, rows_b1, outbuf, fbuf, bbuf,
          sp_s, sp_d, sp_t,
          sem_r0, sem_r1, sem_i0, sem_i1):
    wid = lax.axis_index("s") * NC + lax.axis_index("c")
    idx_bufs = (idx_b0, idx_b1)
    rows_bufs = (rows_b0, rows_b1)
    sem_rows = (sem_r0, sem_r1)
    sem_idx = (sem_i0, sem_i1)

    sid = lax.axis_index("s")
    vrows = VOCAB // NS
    CONV = 125
    for tab_hbm, sp in ((stab, sp_s), (dtab, sp_d), (ttab, sp_t)):
        for k in range(vrows // CONV):
            r0 = sid * vrows + k * CONV
            pltpu.sync_copy(tab_hbm.at[pl.ds(r0, CONV)],
                            fbuf.at[pl.ds(0, CONV)])

            @pl.loop(0, CONV)
            def conv_loop(r):
                a = fbuf[r, pl.ds(0, 16)]
                b = fbuf[r, pl.ds(16, 16)]
                bbuf[r] = plsc.pack(a, b,
                                    format=plsc.PackFormat.INTERLEAVED)

            pltpu.sync_copy(bbuf.at[pl.ds(0, CONV)], sp.at[pl.ds(r0, CONV)])
    plsc.subcore_barrier()

    pltpu.sync_copy(cust_i.at[pl.ds(wid * SAMP, SAMP)], idx_c)
    for h in range(SAMP // 128):
        pltpu.async_copy(ctab.at[idx_c.at[pl.ds(h * 128, 128)]],
                         fbuf.at[pl.ds(0, 128)], sem_r0).wait()

        @pl.loop(0, 128)
        def cust_loop(s):
            outbuf[h * 128 + s, pl.ds(0, 16)] = fbuf[s, pl.ds(0, 16)]
            outbuf[h * 128 + s, pl.ds(16, 16)] = fbuf[s, pl.ds(16, 16)]

    _pooled_field(wid, subj_i, sp_s, EMB, idx_bufs, rows_bufs, outbuf,
                  sem_rows, sem_idx, SUBJ_T, SUBJ_SPC)
    _pooled_field(wid, desc_i, sp_d, 2 * EMB, idx_bufs, rows_bufs, outbuf,
                  sem_rows, sem_idx, DESC_T, DESC_SPC)
    _pooled_field(wid, type_i, sp_t, 3 * EMB, idx_bufs, rows_bufs, outbuf,
                  sem_rows, sem_idx, TYPE_T, TYPE_SPC)

    pltpu.sync_copy(outbuf, out.at[pl.ds(wid * SAMP, SAMP)])


@jax.jit
def _run(cust_i, subj_i, desc_i, type_i, ctab, stab, dtab, ttab):
    mesh = plsc.VectorSubcoreMesh(core_axis_name="c", subcore_axis_name="s",
                                  num_cores=NC, num_subcores=NS)
    fn = pl.kernel(
        _body,
        out_type=jax.ShapeDtypeStruct((B, OUT_D), jnp.float32),
        mesh=mesh,
        compiler_params=pltpu.CompilerParams(use_tc_tiling_on_sc=False,
                                             needs_layout_passes=False),
        scratch_types=[
            pltpu.VMEM((SAMP,), jnp.int32),
            pltpu.VMEM((MAX_ROWS,), jnp.int32),
            pltpu.VMEM((MAX_ROWS,), jnp.int32),
            pltpu.VMEM((MAX_ROWS, EMB), jnp.bfloat16),
            pltpu.VMEM((MAX_ROWS, EMB), jnp.bfloat16),
            pltpu.VMEM((SAMP, OUT_D), jnp.float32),
            pltpu.VMEM((128, EMB), jnp.float32),
            pltpu.VMEM((128, EMB), jnp.bfloat16),
            pltpu.VMEM_SHARED((VOCAB, EMB), jnp.bfloat16),
            pltpu.VMEM_SHARED((VOCAB, EMB), jnp.bfloat16),
            pltpu.VMEM_SHARED((VOCAB, EMB), jnp.bfloat16),
            pltpu.SemaphoreType.DMA,
            pltpu.SemaphoreType.DMA,
            pltpu.SemaphoreType.DMA,
            pltpu.SemaphoreType.DMA,
        ],
    )
    return fn(cust_i, subj_i, desc_i, type_i, ctab, stab, dtab, ttab)


def kernel(customer_name, ticket_subject, ticket_description, ticket_type,
           customer_table, subject_table, description_table, type_table):
    cust_i = customer_name
    subj_i = ticket_subject.reshape(-1)
    desc_i = ticket_description.reshape(-1)
    type_i = ticket_type.reshape(-1)
    return _run(cust_i, subj_i, desc_i, type_i,
                customer_table, subject_table,
                description_table, type_table)
